# Initial kernel scaffold; baseline (speedup 1.0000x reference)
import jax
import jax.numpy as jnp
from jax.experimental import pallas as pl


def kernel(x, edge_index, edge_attr, batch, W_edge, W_gnn, b_gnn, W_imp, b_imp, W_p1, b_p1, W_p2, b_p2):
    raise NotImplementedError("write your pallas kernel here")


# speedup vs baseline: 2.1645x; 2.1645x over previous
import jax
import jax.numpy as jnp
from jax import lax
from jax.experimental import pallas as pl
from jax.experimental.pallas import tpu as pltpu
from jax.experimental.pallas import tpu_sc as plsc

N = 10000
E = 320000
DF = 128
DE = 4
DEP = 16
DH = 300
DHP = 384
G = 128

NC, NS = 2, 16
NW = NC * NS
CHUNK = 128
CPW = 79
EPAD = NW * CPW * CHUNK
NP = 10240
RPT = NP // NS
ZR = 128
EB = 1024


def _sc_body(x_hbm, src_hbm, dst_hbm, e_hbm, outx_hbm,
             aggx_sh, src_v, dst_v, xbuf, ebuf, sem):
    cid = lax.axis_index("c")
    sid = lax.axis_index("s")
    wid = cid * NS + sid

    zeros16 = jnp.zeros((16,), jnp.float32)

    @pl.loop(0, ZR)
    def _zero(i):
        for j in range(DF // 16):
            xbuf[i, pl.ds(j * 16, 16)] = zeros16

    def _ramp(idx_ref, base):
        for j in range(CHUNK // 16):
            idx_ref[pl.ds(j * 16, 16)] = lax.iota(jnp.int32, 16) + (base + j * 16)

    for k in range(RPT // ZR):
        r0 = sid * RPT + k * ZR
        _ramp(src_v, r0)
        pltpu.sync_copy(xbuf, aggx_sh.at[src_v])
    plsc.subcore_barrier()

    @pl.loop(0, CPW)
    def _edges(i):
        base = (wid * CPW + i) * CHUNK
        pltpu.sync_copy(src_hbm.at[pl.ds(base, CHUNK)], src_v)
        pltpu.sync_copy(dst_hbm.at[pl.ds(base, CHUNK)], dst_v)
        pltpu.async_copy(x_hbm.at[src_v], xbuf, sem).wait()
        pltpu.sync_copy(e_hbm.at[pl.ds(base, CHUNK), :], ebuf)
        pltpu.sync_copy(xbuf, aggx_sh.at[dst_v], add=True)
        pltpu.sync_copy(ebuf, aggx_sh.at[dst_v], add=True)

    plsc.subcore_barrier()

    for k in range(RPT // ZR):
        r0 = sid * RPT + k * ZR
        o0 = cid * NP + r0
        _ramp(src_v, r0)
        pltpu.sync_copy(aggx_sh.at[src_v], xbuf)
        pltpu.sync_copy(xbuf, outx_hbm.at[pl.ds(o0, ZR), :])


def _sc_agg(x_pad, src_pad, dst_pad, e_rows):
    return pl.kernel(
        _sc_body,
        out_type=jax.ShapeDtypeStruct((NC * NP, DF), jnp.float32),
        mesh=plsc.VectorSubcoreMesh(core_axis_name="c", subcore_axis_name="s"),
        scratch_types=[
            pltpu.VMEM_SHARED((NP, DF), jnp.float32),
            pltpu.VMEM((CHUNK,), jnp.int32),
            pltpu.VMEM((CHUNK,), jnp.int32),
            pltpu.VMEM((CHUNK, DF), jnp.float32),
            pltpu.VMEM((CHUNK, DF), jnp.float32),
            pltpu.SemaphoreType.DMA,
        ],
    )(x_pad, src_pad, dst_pad, e_rows)


def _edge_mm_body(ea_ref, we_ref, out_ref):
    out_ref[...] = jnp.dot(ea_ref[...], we_ref[...],
                           preferred_element_type=jnp.float32)


_edge_mm = pl.pallas_call(
    _edge_mm_body,
    grid=(EPAD // EB,),
    in_specs=[
        pl.BlockSpec((EB, DEP), lambda i: (i, 0)),
        pl.BlockSpec((DEP, DF), lambda i: (0, 0)),
    ],
    out_specs=pl.BlockSpec((EB, DF), lambda i: (i, 0)),
    out_shape=jax.ShapeDtypeStruct((EPAD, DF), jnp.float32),
)


def _tc_body(paggx, x, batch, W_gnn, b_gnn, W_imp, b_imp,
             W_p1, b_p1, W_p2, b_p2, out_ref):
    f32 = jnp.float32
    t = paggx[0:N, :] + paggx[NP:NP + N, :] + x[...]

    imp_pre = jnp.dot(t, W_imp[...], preferred_element_type=f32) + b_imp[0, 0]
    imp = 1.0 / (1.0 + jnp.exp(-imp_pre))
    impT_pre = lax.dot_general(W_imp[...], t, (((0,), (1,)), ((), ())),
                               preferred_element_type=f32) + b_imp[0, 0]
    impT = 1.0 / (1.0 + jnp.exp(-impT_pre))

    h = jnp.maximum(jnp.dot(t, W_gnn[...], preferred_element_type=f32)
                    + b_gnn[...], 0.0)

    gid = lax.broadcasted_iota(jnp.int32, (G, N), 0)
    oh = jnp.broadcast_to(batch[...], (G, N)) == gid
    ohf = oh.astype(f32)

    impb = jnp.broadcast_to(impT[0:1, :], (G, N))
    m10 = jnp.max(jnp.where(oh, impb, -1.0), axis=1, keepdims=True) * 10.0
    node_m = lax.dot_general(ohf, m10, (((0,), (0,)), ((), ())),
                             preferred_element_type=f32)
    wgt = imp[:, 0:1] / node_m + 0.9

    hw = h * jnp.broadcast_to(wgt, (N, DHP))
    sums = jnp.dot(ohf, hw, preferred_element_type=f32)
    counts = jnp.sum(ohf, axis=1, keepdims=True)
    pooled = sums / jnp.maximum(counts, 1.0)

    hid = jnp.maximum(jnp.dot(pooled, W_p1[...], preferred_element_type=f32)
                      + b_p1[...], 0.0)
    out_ref[...] = jnp.dot(hid, W_p2[...], preferred_element_type=f32) + b_p2[...]


_tc_call = pl.pallas_call(
    _tc_body,
    out_shape=jax.ShapeDtypeStruct((G, DHP), jnp.float32),
)


def kernel(x, edge_index, edge_attr, batch, W_edge, W_gnn, b_gnn, W_imp, b_imp,
           W_p1, b_p1, W_p2, b_p2):
    src = edge_index[0]
    dst = edge_index[1]

    pad = EPAD - E
    x_pad = jnp.concatenate([x, jnp.zeros((8, DF), jnp.float32)], axis=0)
    src_pad = jnp.concatenate([src, jnp.full((pad,), N, jnp.int32)])
    dst_pad = jnp.concatenate([dst, jnp.zeros((pad,), jnp.int32)])
    ea_pad = jnp.pad(edge_attr, ((0, pad), (0, DEP - DE)))
    W_edge_p = jnp.pad(W_edge, ((0, DEP - DE), (0, 0)))

    e_rows = _edge_mm(ea_pad, W_edge_p)
    paggx = _sc_agg(x_pad, src_pad, dst_pad, e_rows)

    W_gnn_p = jnp.pad(W_gnn, ((0, 0), (0, DHP - DH)))
    b_gnn_p = jnp.pad(b_gnn, (0, DHP - DH)).reshape(1, DHP)
    W_imp_p = jnp.broadcast_to(W_imp, (DF, 8))
    b_imp_p = b_imp.reshape(1, 1)
    W_p1_p = jnp.pad(W_p1, ((0, DHP - DH), (0, DHP - DH)))
    b_p1_p = jnp.pad(b_p1, (0, DHP - DH)).reshape(1, DHP)
    W_p2_p = jnp.pad(W_p2, ((0, DHP - DH), (0, DHP - DH)))
    b_p2_p = jnp.pad(b_p2, (0, DHP - DH)).reshape(1, DHP)
    batch2 = batch.reshape(1, N)

    z = _tc_call(paggx, x, batch2, W_gnn_p, b_gnn_p,
                 W_imp_p, b_imp_p, W_p1_p, b_p1_p, W_p2_p, b_p2_p)
    return z[:, :DH]


# speedup vs baseline: 2.5668x; 1.1859x over previous
import jax
import jax.numpy as jnp
from jax import lax
from jax.experimental import pallas as pl
from jax.experimental.pallas import tpu as pltpu
from jax.experimental.pallas import tpu_sc as plsc

N = 10000
E = 320000
DF = 128
DE = 4
DEP = 16
DH = 300
DHP = 384
G = 128

NC, NS = 2, 16
NW = NC * NS
CHUNK = 64
CPW = 158
EPAD = NW * CPW * CHUNK
NP = 10240
RPT = NP // NS
ZR = 64
EB = 1024


def _sc_body(x_hbm, src_hbm, dst_hbm, e_hbm, outx_hbm,
             aggx_sh, src0, src1, dst0, dst1, xbuf0, xbuf1, ebuf0, ebuf1,
             xsem0, xsem1, esem0, esem1):
    cid = lax.axis_index("c")
    sid = lax.axis_index("s")
    wid = cid * NS + sid
    srcs, dsts = (src0, src1), (dst0, dst1)
    xbufs, ebufs = (xbuf0, xbuf1), (ebuf0, ebuf1)
    xsems, esems = (xsem0, xsem1), (esem0, esem1)

    zeros16 = jnp.zeros((16,), jnp.float32)

    @pl.loop(0, ZR)
    def _zero(i):
        for j in range(DF // 16):
            xbuf0[i, pl.ds(j * 16, 16)] = zeros16

    def _ramp(idx_ref, base):
        for j in range(CHUNK // 16):
            idx_ref[pl.ds(j * 16, 16)] = lax.iota(jnp.int32, 16) + (base + j * 16)

    for k in range(RPT // ZR):
        r0 = sid * RPT + k * ZR
        _ramp(src0, r0)
        pltpu.sync_copy(xbuf0, aggx_sh.at[src0])
    plsc.subcore_barrier()

    def _fire(c, b):
        base = (wid * CPW + c) * CHUNK
        pltpu.sync_copy(src_hbm.at[pl.ds(base, CHUNK)], srcs[b])
        pltpu.sync_copy(dst_hbm.at[pl.ds(base, CHUNK)], dsts[b])
        pltpu.async_copy(x_hbm.at[srcs[b]], xbufs[b], xsems[b])
        pltpu.async_copy(e_hbm.at[pl.ds(base, CHUNK), :], ebufs[b], esems[b])

    def _drain_and_add(c, b):
        base = (wid * CPW + c) * CHUNK
        pltpu.make_async_copy(x_hbm.at[srcs[b]], xbufs[b], xsems[b]).wait()
        pltpu.make_async_copy(
            e_hbm.at[pl.ds(base, CHUNK), :], ebufs[b], esems[b]).wait()
        pltpu.sync_copy(xbufs[b], aggx_sh.at[dsts[b]], add=True)
        pltpu.sync_copy(ebufs[b], aggx_sh.at[dsts[b]], add=True)

    _fire(0, 0)

    @pl.loop(0, CPW // 2)
    def _edges(g):
        for b in range(2):
            c = g * 2 + b

            @pl.when(c + 1 < CPW)
            def _():
                _fire(c + 1, 1 - b)

            _drain_and_add(c, b)

    plsc.subcore_barrier()

    for k in range(RPT // ZR):
        r0 = sid * RPT + k * ZR
        o0 = cid * NP + r0
        _ramp(src0, r0)
        pltpu.sync_copy(aggx_sh.at[src0], xbuf0)
        pltpu.sync_copy(xbuf0, outx_hbm.at[pl.ds(o0, ZR), :])


def _sc_agg(x_pad, src_pad, dst_pad, e_rows):
    return pl.kernel(
        _sc_body,
        out_type=jax.ShapeDtypeStruct((NC * NP, DF), jnp.float32),
        mesh=plsc.VectorSubcoreMesh(core_axis_name="c", subcore_axis_name="s"),
        scratch_types=[
            pltpu.VMEM_SHARED((NP, DF), jnp.float32),
            pltpu.VMEM((CHUNK,), jnp.int32),
            pltpu.VMEM((CHUNK,), jnp.int32),
            pltpu.VMEM((CHUNK,), jnp.int32),
            pltpu.VMEM((CHUNK,), jnp.int32),
            pltpu.VMEM((CHUNK, DF), jnp.float32),
            pltpu.VMEM((CHUNK, DF), jnp.float32),
            pltpu.VMEM((CHUNK, DF), jnp.float32),
            pltpu.VMEM((CHUNK, DF), jnp.float32),
            pltpu.SemaphoreType.DMA,
            pltpu.SemaphoreType.DMA,
            pltpu.SemaphoreType.DMA,
            pltpu.SemaphoreType.DMA,
        ],
    )(x_pad, src_pad, dst_pad, e_rows)


def _edge_mm_body(ea_ref, we_ref, out_ref):
    out_ref[...] = jnp.dot(ea_ref[...], we_ref[...],
                           preferred_element_type=jnp.float32)


_edge_mm = pl.pallas_call(
    _edge_mm_body,
    grid=(EPAD // EB,),
    in_specs=[
        pl.BlockSpec((EB, DEP), lambda i: (i, 0)),
        pl.BlockSpec((DEP, DF), lambda i: (0, 0)),
    ],
    out_specs=pl.BlockSpec((EB, DF), lambda i: (i, 0)),
    out_shape=jax.ShapeDtypeStruct((EPAD, DF), jnp.float32),
)


def _tc_body(paggx, x, batch, W_gnn, b_gnn, W_imp, b_imp,
             W_p1, b_p1, W_p2, b_p2, out_ref):
    f32 = jnp.float32
    t = paggx[0:N, :] + paggx[NP:NP + N, :] + x[...]

    imp_pre = jnp.dot(t, W_imp[...], preferred_element_type=f32) + b_imp[0, 0]
    imp = 1.0 / (1.0 + jnp.exp(-imp_pre))
    impT_pre = lax.dot_general(W_imp[...], t, (((0,), (1,)), ((), ())),
                               preferred_element_type=f32) + b_imp[0, 0]
    impT = 1.0 / (1.0 + jnp.exp(-impT_pre))

    h = jnp.maximum(jnp.dot(t, W_gnn[...], preferred_element_type=f32)
                    + b_gnn[...], 0.0)

    gid = lax.broadcasted_iota(jnp.int32, (G, N), 0)
    oh = jnp.broadcast_to(batch[...], (G, N)) == gid
    ohf = oh.astype(f32)

    impb = jnp.broadcast_to(impT[0:1, :], (G, N))
    m10 = jnp.max(jnp.where(oh, impb, -1.0), axis=1, keepdims=True) * 10.0
    node_m = lax.dot_general(ohf, m10, (((0,), (0,)), ((), ())),
                             preferred_element_type=f32)
    wgt = imp[:, 0:1] / node_m + 0.9

    hw = h * jnp.broadcast_to(wgt, (N, DHP))
    sums = jnp.dot(ohf, hw, preferred_element_type=f32)
    counts = jnp.sum(ohf, axis=1, keepdims=True)
    pooled = sums / jnp.maximum(counts, 1.0)

    hid = jnp.maximum(jnp.dot(pooled, W_p1[...], preferred_element_type=f32)
                      + b_p1[...], 0.0)
    out_ref[...] = jnp.dot(hid, W_p2[...], preferred_element_type=f32) + b_p2[...]


_tc_call = pl.pallas_call(
    _tc_body,
    out_shape=jax.ShapeDtypeStruct((G, DHP), jnp.float32),
)


def kernel(x, edge_index, edge_attr, batch, W_edge, W_gnn, b_gnn, W_imp, b_imp,
           W_p1, b_p1, W_p2, b_p2):
    src = edge_index[0]
    dst = edge_index[1]

    pad = EPAD - E
    x_pad = jnp.concatenate([x, jnp.zeros((8, DF), jnp.float32)], axis=0)
    src_pad = jnp.concatenate([src, jnp.full((pad,), N, jnp.int32)])
    dst_pad = jnp.concatenate([dst, jnp.zeros((pad,), jnp.int32)])
    ea_pad = jnp.pad(edge_attr, ((0, pad), (0, DEP - DE)))
    W_edge_p = jnp.pad(W_edge, ((0, DEP - DE), (0, 0)))

    e_rows = _edge_mm(ea_pad, W_edge_p)
    paggx = _sc_agg(x_pad, src_pad, dst_pad, e_rows)

    W_gnn_p = jnp.pad(W_gnn, ((0, 0), (0, DHP - DH)))
    b_gnn_p = jnp.pad(b_gnn, (0, DHP - DH)).reshape(1, DHP)
    W_imp_p = jnp.broadcast_to(W_imp, (DF, 8))
    b_imp_p = b_imp.reshape(1, 1)
    W_p1_p = jnp.pad(W_p1, ((0, DHP - DH), (0, DHP - DH)))
    b_p1_p = jnp.pad(b_p1, (0, DHP - DH)).reshape(1, DHP)
    W_p2_p = jnp.pad(W_p2, ((0, DHP - DH), (0, DHP - DH)))
    b_p2_p = jnp.pad(b_p2, (0, DHP - DH)).reshape(1, DHP)
    batch2 = batch.reshape(1, N)

    z = _tc_call(paggx, x, batch2, W_gnn_p, b_gnn_p,
                 W_imp_p, b_imp_p, W_p1_p, b_p1_p, W_p2_p, b_p2_p)
    return z[:, :DH]
